# Initial kernel scaffold; baseline (speedup 1.0000x reference)
import jax
import jax.numpy as jnp
from jax.experimental import pallas as pl


def kernel(output):
    raise NotImplementedError("write your pallas kernel here")


# speedup vs baseline: 31.1578x; 31.1578x over previous
import functools

import jax
import jax.numpy as jnp
from jax.experimental import pallas as pl
from jax.experimental.pallas import tpu as pltpu

_IOU_THR = 0.2
_EPS = 1e-10
_N = 5000
_L = 128
_R = 40
_NP = _R * _L


def _nms_body(l_s, t_s, r_s, b_s, a_s, l_v, t_v, r_v, b_v, a_v,
              keep_ref, active_ref):
    row = jax.lax.broadcasted_iota(jnp.int32, (_R, _L), 0)
    col = jax.lax.broadcasted_iota(jnp.int32, (_R, _L), 1)
    fidx = row * _L + col

    active_ref[...] = jnp.ones((_R, _L), jnp.float32)
    keep_ref[...] = jnp.zeros((_R, _L), jnp.float32)

    lv = l_v[...]
    tv = t_v[...]
    rv = r_v[...]
    bv = b_v[...]
    av = a_v[...]

    def body(i, carry):
        act = active_ref[...]
        is_act = jnp.sum(jnp.where(fidx == i, act, 0.0))
        ai = a_s[i]

        @pl.when((is_act > 0.0) & (ai >= 4.0))
        def _():
            li = l_s[i]
            ti = t_s[i]
            ri = r_s[i]
            bi = b_s[i]
            iw = jnp.maximum(jnp.minimum(rv, ri) - jnp.maximum(lv, li), 0.0)
            ih = jnp.maximum(jnp.minimum(bv, bi) - jnp.maximum(tv, ti), 0.0)
            inter = iw * ih
            union = (ai + av) - inter + _EPS
            iou = inter / union
            supp = (act > 0.0) & (fidx > i) & (iou >= _IOU_THR)
            cnt = jnp.sum(jnp.where(supp, 1.0, 0.0))

            @pl.when(cnt >= 10.0)
            def _():
                active_ref[...] = jnp.where(supp, 0.0, act)
                keep_ref[...] = jnp.where(fidx == i, 1.0, keep_ref[...])

        return carry

    jax.lax.fori_loop(0, _N, body, 0)


@jax.jit
def kernel(output):
    n, c = output.shape
    order = jnp.argsort(-output[:, 0])
    out = output[order]

    l = out[:, 1]
    t = out[:, 2]
    r = out[:, 3]
    b = out[:, 4]
    a = (r - l) * (b - t)

    pad = jnp.zeros((_NP - n,), jnp.float32)

    def p2(x):
        return jnp.concatenate([x, pad]).reshape(_R, _L)

    def p1(x):
        return jnp.concatenate([x, pad])

    smem_spec = pl.BlockSpec(memory_space=pltpu.SMEM)
    vmem_spec = pl.BlockSpec(memory_space=pltpu.VMEM)

    keep2d = pl.pallas_call(
        _nms_body,
        out_shape=jax.ShapeDtypeStruct((_R, _L), jnp.float32),
        in_specs=[smem_spec] * 5 + [vmem_spec] * 5,
        out_specs=vmem_spec,
        scratch_shapes=[pltpu.VMEM((_R, _L), jnp.float32)],
    )(p1(l), p1(t), p1(r), p1(b), p1(a),
      p2(l), p2(t), p2(r), p2(b), p2(a))

    keep = keep2d.reshape(_NP)[:n]
    return out * keep[:, None]


# speedup vs baseline: 57.6586x; 1.8505x over previous
import functools

import jax
import jax.numpy as jnp
from jax.experimental import pallas as pl
from jax.experimental.pallas import tpu as pltpu

_IOU_THR = 0.2
_EPS = 1e-10
_N = 5000
_L = 128
_R = 40
_NP = _R * _L


_BIG = 65536.0


def _nms_body(l_s, t_s, r_s, b_s, a_s, l_v, t_v, r_v, b_v, a_v,
              keep_ref, active_ref):
    row = jax.lax.broadcasted_iota(jnp.int32, (_R, _L), 0)
    col = jax.lax.broadcasted_iota(jnp.int32, (_R, _L), 1)
    fidx = row * _L + col

    active_ref[...] = jnp.ones((_R, _L), jnp.float32)
    keep_ref[...] = jnp.zeros((_R, _L), jnp.float32)

    lv = l_v[...]
    tv = t_v[...]
    rv = r_v[...]
    bv = b_v[...]
    av = a_v[...]

    def body(i, carry):
        act = active_ref[...]
        li = l_s[i]
        ti = t_s[i]
        ri = r_s[i]
        bi = b_s[i]
        ai = a_s[i]
        iw = jnp.maximum(jnp.minimum(rv, ri) - jnp.maximum(lv, li), 0.0)
        ih = jnp.maximum(jnp.minimum(bv, bi) - jnp.maximum(tv, ti), 0.0)
        inter = iw * ih
        union = (ai + av) - inter + _EPS
        iou = inter / union
        onehot = fidx == i
        supp = jnp.where((iou >= _IOU_THR) & (fidx > i), act, 0.0)
        combined = supp + jnp.where(onehot, act * _BIG, 0.0)
        s = jnp.sum(jnp.sum(combined, axis=0, keepdims=True),
                    axis=1, keepdims=True)
        okv = (s >= _BIG + 10.0) & (ai >= 4.0)
        okf = jnp.where(okv, 1.0, 0.0)
        active_ref[...] = act - okf * supp
        keep_ref[...] = keep_ref[...] + jnp.where(onehot, okf, 0.0)
        return carry

    jax.lax.fori_loop(0, _N, body, 0)


@jax.jit
def kernel(output):
    n, c = output.shape
    order = jnp.argsort(-output[:, 0])
    out = output[order]

    l = out[:, 1]
    t = out[:, 2]
    r = out[:, 3]
    b = out[:, 4]
    a = (r - l) * (b - t)

    pad = jnp.zeros((_NP - n,), jnp.float32)

    def p2(x):
        return jnp.concatenate([x, pad]).reshape(_R, _L)

    def p1(x):
        return jnp.concatenate([x, pad])

    smem_spec = pl.BlockSpec(memory_space=pltpu.SMEM)
    vmem_spec = pl.BlockSpec(memory_space=pltpu.VMEM)

    keep2d = pl.pallas_call(
        _nms_body,
        out_shape=jax.ShapeDtypeStruct((_R, _L), jnp.float32),
        in_specs=[smem_spec] * 5 + [vmem_spec] * 5,
        out_specs=vmem_spec,
        scratch_shapes=[pltpu.VMEM((_R, _L), jnp.float32)],
    )(p1(l), p1(t), p1(r), p1(b), p1(a),
      p2(l), p2(t), p2(r), p2(b), p2(a))

    keep = keep2d.reshape(_NP)[:n]
    return out * keep[:, None]


# speedup vs baseline: 141.3047x; 2.4507x over previous
import functools

import jax
import jax.numpy as jnp
from jax.experimental import pallas as pl
from jax.experimental.pallas import tpu as pltpu

_IOU_THR = 0.2
_EPS = 1e-10
_N = 5000
_L = 128
_R = 40
_NP = _R * _L


_BIG = 8192.0


_C = 8


def _nms_body(l_s, t_s, r_s, b_s, a_s, l_v, t_v, r_v, b_v, a_v, keep_ref):
    row = jax.lax.broadcasted_iota(jnp.int32, (_R, _L), 0)
    col = jax.lax.broadcasted_iota(jnp.int32, (_R, _L), 1)
    fidx = row * _L + col

    lv = l_v[...]
    tv = t_v[...]
    rv = r_v[...]
    bv = b_v[...]
    av = a_v[...]

    def cond(carry):
        return carry[0] < _N

    def step(carry):
        p, act, keep = carry
        supps = []
        oks = []
        for c in range(_C):
            ic = p + c
            li = l_s[ic]
            ti = t_s[ic]
            ri = r_s[ic]
            bi = b_s[ic]
            ai = a_s[ic]
            iw = jnp.maximum(jnp.minimum(rv, ri) - jnp.maximum(lv, li), 0.0)
            ih = jnp.maximum(jnp.minimum(bv, bi) - jnp.maximum(tv, ti), 0.0)
            inter = iw * ih
            union = (ai + av) - inter + _EPS
            iou = inter / union
            supp = jnp.where((iou >= _IOU_THR) & (fidx > ic), act, 0.0)
            combined = supp + jnp.where(fidx == ic, act * _BIG, 0.0)
            s = jnp.sum(combined)
            oks.append((s >= _BIG + 10.0) & (ai >= 4.0))
            supps.append(supp)

        first = jnp.int32(_C)
        for c in reversed(range(_C)):
            first = jnp.where(oks[c], jnp.int32(c), first)

        kept_f = jnp.where(first < _C, 1.0, 0.0)
        keep = keep + jnp.where(fidx == p + first, kept_f, 0.0)
        for c in range(_C):
            act = act - jnp.where(first == c, 1.0, 0.0) * supps[c]
        p = p + jnp.minimum(first + 1, _C)
        return p, act, keep

    act0 = jnp.ones((_R, _L), jnp.float32)
    keep0 = jnp.zeros((_R, _L), jnp.float32)
    _, _, keep = jax.lax.while_loop(cond, step, (jnp.int32(0), act0, keep0))
    keep_ref[...] = keep


@jax.jit
def kernel(output):
    n, c = output.shape
    order = jnp.argsort(-output[:, 0])
    out = output[order]

    l = out[:, 1]
    t = out[:, 2]
    r = out[:, 3]
    b = out[:, 4]
    a = (r - l) * (b - t)

    pad = jnp.zeros((_NP - n,), jnp.float32)

    def p2(x):
        return jnp.concatenate([x, pad]).reshape(_R, _L)

    def p1(x):
        return jnp.concatenate([x, pad])

    smem_spec = pl.BlockSpec(memory_space=pltpu.SMEM)
    vmem_spec = pl.BlockSpec(memory_space=pltpu.VMEM)

    keep2d = pl.pallas_call(
        _nms_body,
        out_shape=jax.ShapeDtypeStruct((_R, _L), jnp.float32),
        in_specs=[smem_spec] * 5 + [vmem_spec] * 5,
        out_specs=vmem_spec,
    )(p1(l), p1(t), p1(r), p1(b), p1(a),
      p2(l), p2(t), p2(r), p2(b), p2(a))

    keep = keep2d.reshape(_NP)[:n]
    return out * keep[:, None]


# speedup vs baseline: 153.6980x; 1.0877x over previous
import functools

import jax
import jax.numpy as jnp
from jax.experimental import pallas as pl
from jax.experimental.pallas import tpu as pltpu

_IOU_THR = 0.2
_EPS = 1e-10
_N = 5000
_L = 128
_R = 40
_NP = _R * _L


_BIG = 8192.0


_C = 16


def _nms_body(l_s, t_s, r_s, b_s, a_s, l_v, t_v, r_v, b_v, a_v, keep_ref):
    row = jax.lax.broadcasted_iota(jnp.int32, (_R, _L), 0)
    col = jax.lax.broadcasted_iota(jnp.int32, (_R, _L), 1)
    fidx = row * _L + col

    lv = l_v[...]
    tv = t_v[...]
    rv = r_v[...]
    bv = b_v[...]
    av = a_v[...]

    def cond(carry):
        return carry[0] < _N

    def step(carry):
        p, act, keep = carry
        supps = []
        oks = []
        for c in range(_C):
            ic = p + c
            li = l_s[ic]
            ti = t_s[ic]
            ri = r_s[ic]
            bi = b_s[ic]
            ai = a_s[ic]
            iw = jnp.maximum(jnp.minimum(rv, ri) - jnp.maximum(lv, li), 0.0)
            ih = jnp.maximum(jnp.minimum(bv, bi) - jnp.maximum(tv, ti), 0.0)
            inter = iw * ih
            union = (ai + av) - inter + _EPS
            iou = inter / union
            supp = jnp.where((iou >= _IOU_THR) & (fidx > ic), act, 0.0)
            combined = supp + jnp.where(fidx == ic, act * _BIG, 0.0)
            s = jnp.sum(combined)
            oks.append((s >= _BIG + 10.0) & (ai >= 4.0))
            supps.append(supp)

        first = jnp.int32(_C)
        for c in reversed(range(_C)):
            first = jnp.where(oks[c], jnp.int32(c), first)

        kept_f = jnp.where(first < _C, 1.0, 0.0)
        keep = keep + jnp.where(fidx == p + first, kept_f, 0.0)
        for c in range(_C):
            act = act - jnp.where(first == c, 1.0, 0.0) * supps[c]
        p = p + jnp.minimum(first + 1, _C)
        return p, act, keep

    act0 = jnp.ones((_R, _L), jnp.float32)
    keep0 = jnp.zeros((_R, _L), jnp.float32)
    _, _, keep = jax.lax.while_loop(cond, step, (jnp.int32(0), act0, keep0))
    keep_ref[...] = keep


@jax.jit
def kernel(output):
    n, c = output.shape
    order = jnp.argsort(-output[:, 0])
    out = output[order]

    l = out[:, 1]
    t = out[:, 2]
    r = out[:, 3]
    b = out[:, 4]
    a = (r - l) * (b - t)

    pad = jnp.zeros((_NP - n,), jnp.float32)

    def p2(x):
        return jnp.concatenate([x, pad]).reshape(_R, _L)

    def p1(x):
        return jnp.concatenate([x, pad])

    smem_spec = pl.BlockSpec(memory_space=pltpu.SMEM)
    vmem_spec = pl.BlockSpec(memory_space=pltpu.VMEM)

    keep2d = pl.pallas_call(
        _nms_body,
        out_shape=jax.ShapeDtypeStruct((_R, _L), jnp.float32),
        in_specs=[smem_spec] * 5 + [vmem_spec] * 5,
        out_specs=vmem_spec,
    )(p1(l), p1(t), p1(r), p1(b), p1(a),
      p2(l), p2(t), p2(r), p2(b), p2(a))

    keep = keep2d.reshape(_NP)[:n]
    return out * keep[:, None]


# speedup vs baseline: 161.5938x; 1.0514x over previous
import functools

import jax
import jax.numpy as jnp
from jax.experimental import pallas as pl
from jax.experimental.pallas import tpu as pltpu

_IOU_THR = 0.2
_EPS = 1e-10
_N = 5000
_L = 128
_R = 40
_NP = _R * _L


_BIG = 8192.0


_C = 16


def _nms_body(l_s, t_s, r_s, b_s, a_s, l_v, t_v, r_v, b_v, a_v, keep_ref):
    row = jax.lax.broadcasted_iota(jnp.int32, (_R, _L), 0)
    col = jax.lax.broadcasted_iota(jnp.int32, (_R, _L), 1)
    fidx = row * _L + col

    lv = l_v[...]
    tv = t_v[...]
    rv = r_v[...]
    bv = b_v[...]
    av = a_v[...]

    def cond(carry):
        return carry[0] < _N

    iota_c = jax.lax.broadcasted_iota(jnp.int32, (_C, 1), 0)

    def step(carry):
        p, act, keep = carry
        supps = []
        srows = []
        for c in range(_C):
            ic = p + c
            li = l_s[ic]
            ti = t_s[ic]
            ri = r_s[ic]
            bi = b_s[ic]
            ai = a_s[ic]
            bigw = jnp.where(ai >= 4.0, _BIG, 0.0)
            iw = jnp.maximum(jnp.minimum(rv, ri) - jnp.maximum(lv, li), 0.0)
            ih = jnp.maximum(jnp.minimum(bv, bi) - jnp.maximum(tv, ti), 0.0)
            inter = iw * ih
            union = (ai + av) - inter + _EPS
            iou = inter / union
            supp = jnp.where((iou >= _IOU_THR) & (fidx > ic), act, 0.0)
            combined = supp + jnp.where(fidx == ic, act * bigw, 0.0)
            srows.append(jnp.sum(combined, axis=0, keepdims=True))
            supps.append(supp)

        s_all = jnp.sum(jnp.concatenate(srows, axis=0), axis=1, keepdims=True)
        ok_all = s_all >= _BIG + 10.0
        firstv = jnp.min(jnp.where(ok_all, iota_c, _C), axis=0, keepdims=True)
        kept_f = jnp.where(firstv < _C, 1.0, 0.0)
        keep = keep + jnp.where(fidx == p + firstv, kept_f, 0.0)
        for c in range(_C):
            act = act - jnp.where(firstv == c, 1.0, 0.0) * supps[c]
        first = jnp.min(jnp.where(ok_all, iota_c, _C))
        p = p + jnp.minimum(first + 1, _C)
        return p, act, keep

    act0 = jnp.ones((_R, _L), jnp.float32)
    keep0 = jnp.zeros((_R, _L), jnp.float32)
    _, _, keep = jax.lax.while_loop(cond, step, (jnp.int32(0), act0, keep0))
    keep_ref[...] = keep


@jax.jit
def kernel(output):
    n, c = output.shape
    order = jnp.argsort(-output[:, 0])
    out = output[order]

    l = out[:, 1]
    t = out[:, 2]
    r = out[:, 3]
    b = out[:, 4]
    a = (r - l) * (b - t)

    pad = jnp.zeros((_NP - n,), jnp.float32)

    def p2(x):
        return jnp.concatenate([x, pad]).reshape(_R, _L)

    def p1(x):
        return jnp.concatenate([x, pad])

    smem_spec = pl.BlockSpec(memory_space=pltpu.SMEM)
    vmem_spec = pl.BlockSpec(memory_space=pltpu.VMEM)

    keep2d = pl.pallas_call(
        _nms_body,
        out_shape=jax.ShapeDtypeStruct((_R, _L), jnp.float32),
        in_specs=[smem_spec] * 5 + [vmem_spec] * 5,
        out_specs=vmem_spec,
    )(p1(l), p1(t), p1(r), p1(b), p1(a),
      p2(l), p2(t), p2(r), p2(b), p2(a))

    keep = keep2d.reshape(_NP)[:n]
    return out * keep[:, None]


# speedup vs baseline: 187.2712x; 1.1589x over previous
import functools

import jax
import jax.numpy as jnp
from jax.experimental import pallas as pl
from jax.experimental.pallas import tpu as pltpu

_IOU_THR = 0.2
_EPS = 1e-10
_N = 5000
_L = 128
_R = 40
_NP = _R * _L


_BIG = 8192.0


_C = 16


def _nms_body(l_s, t_s, r_s, b_s, a_s, l_v, t_v, r_v, b_v, a_v, keep_ref):
    row = jax.lax.broadcasted_iota(jnp.int32, (_R, _L), 0)
    col = jax.lax.broadcasted_iota(jnp.int32, (_R, _L), 1)
    fidx = row * _L + col

    lv_f = l_v[...]
    tv_f = t_v[...]
    rv_f = r_v[...]
    bv_f = b_v[...]
    av_f = a_v[...]

    iota_c = jax.lax.broadcasted_iota(jnp.int32, (_C, 1), 0)

    def make_step(start):
        rows = _R - start
        lv = lv_f[start:]
        tv = tv_f[start:]
        rv = rv_f[start:]
        bv = bv_f[start:]
        av = av_f[start:]
        fidxw = fidx[start:]

        def step(carry):
            p, act, keep = carry
            supps = []
            srows = []
            for c in range(_C):
                ic = p + c
                li = l_s[ic]
                ti = t_s[ic]
                ri = r_s[ic]
                bi = b_s[ic]
                ai = a_s[ic]
                bigw = jnp.where(ai >= 4.0, _BIG, 0.0)
                iw = jnp.maximum(jnp.minimum(rv, ri) - jnp.maximum(lv, li),
                                 0.0)
                ih = jnp.maximum(jnp.minimum(bv, bi) - jnp.maximum(tv, ti),
                                 0.0)
                inter = iw * ih
                union = (ai + av) - inter + _EPS
                iou = inter / union
                supp = jnp.where((iou >= _IOU_THR) & (fidxw > ic), act, 0.0)
                combined = supp + jnp.where(fidxw == ic, act * bigw, 0.0)
                srows.append(jnp.sum(combined, axis=0, keepdims=True))
                supps.append(supp)

            s_all = jnp.sum(jnp.concatenate(srows, axis=0), axis=1,
                            keepdims=True)
            ok_all = s_all >= _BIG + 10.0
            firstv = jnp.min(jnp.where(ok_all, iota_c, _C), axis=0,
                             keepdims=True)
            kept_f = jnp.where(firstv < _C, 1.0, 0.0)
            keep = keep + jnp.where(fidxw == p + firstv, kept_f, 0.0)
            sel = supps
            for bit in (1, 2, 4, 8):
                nxt = []
                for j in range(0, len(sel), 2):
                    pred = (firstv & bit) != 0
                    nxt.append(jnp.where(pred, sel[j + 1], sel[j]))
                sel = nxt
            act = act - kept_f * sel[0]
            first = jnp.min(jnp.where(ok_all, iota_c, _C))
            p = p + jnp.minimum(first + 1, _C)
            return p, act, keep

        return step

    phases = [(0, 1024), (8, 2048), (16, 3072), (24, 4096), (32, _N)]
    act = jnp.ones((_R, _L), jnp.float32)
    keep = jnp.zeros((_R, _L), jnp.float32)
    p = jnp.int32(0)
    for k, (start, limit) in enumerate(phases):
        lim = jnp.int32(limit)
        p, act, keep = jax.lax.while_loop(
            lambda carry, lim=lim: carry[0] < lim,
            make_step(start), (p, act, keep))
        if k + 1 < len(phases):
            nstart = phases[k + 1][0]
            delta = nstart - start
            keep_ref[start:nstart, :] = keep[:delta]
            act = act[delta:]
            keep = keep[delta:]
        else:
            keep_ref[start:, :] = keep


@jax.jit
def kernel(output):
    n, c = output.shape
    order = jnp.argsort(-output[:, 0])
    out = output[order]

    l = out[:, 1]
    t = out[:, 2]
    r = out[:, 3]
    b = out[:, 4]
    a = (r - l) * (b - t)

    pad = jnp.zeros((_NP - n,), jnp.float32)

    def p2(x):
        return jnp.concatenate([x, pad]).reshape(_R, _L)

    def p1(x):
        return jnp.concatenate([x, pad])

    smem_spec = pl.BlockSpec(memory_space=pltpu.SMEM)
    vmem_spec = pl.BlockSpec(memory_space=pltpu.VMEM)

    keep2d = pl.pallas_call(
        _nms_body,
        out_shape=jax.ShapeDtypeStruct((_R, _L), jnp.float32),
        in_specs=[smem_spec] * 5 + [vmem_spec] * 5,
        out_specs=vmem_spec,
    )(p1(l), p1(t), p1(r), p1(b), p1(a),
      p2(l), p2(t), p2(r), p2(b), p2(a))

    keep = keep2d.reshape(_NP)[:n]
    return out * keep[:, None]
